# baseline (speedup 1.0000x reference)
import functools
import math

import jax
import jax.numpy as jnp
import numpy as np
from jax import lax
from jax.experimental import pallas as pl
from jax.experimental.pallas import tpu as pltpu
from jax.experimental.pallas import tpu_sc as plsc

B, L, DM, H = 4, 2048, 1024, 16
LEFT, RIGHT = 410, 1025
F = RIGHT - LEFT
FP = 640
TOPK = int(1 * math.log(L))
EPS = 1e-12
CBLK = 256
_INTERP = False

_HI = jax.lax.Precision.HIGHEST
_COR = jax.lax.Precision.HIGHEST
_FAST = jax.lax.Precision.DEFAULT


def _build_consts():
    t = np.arange(L, dtype=np.int64)
    f = np.arange(FP, dtype=np.int64) + LEFT
    ang = 2.0 * np.pi * ((f[:, None] * t[None, :]) % L) / L
    cst = np.cos(ang)
    snt = np.sin(ang)
    cst[F:, :] = 0.0
    snt[F:, :] = 0.0
    coef = np.zeros((FP, 1))
    coef[:F] = 2.0
    coef[F - 1] = 1.0
    wcm = coef * np.cos(ang) / (L * DM)
    wsm = -coef * np.sin(ang) / (L * DM)
    wsm[F - 1, :] = 0.0
    wcm[F:, :] = 0.0
    trig = np.stack([cst, snt]).astype(np.float32)
    wcs = np.concatenate([wcm, wsm], 0).astype(np.float32)
    return trig, wcs


_TRIG, _WCS = _build_consts()


def _prep_body(wq_ref, wk_ref, wv_ref, wd_ref, bv_ref, bd_ref,
               mqk_ref, wvd_ref, bias_ref):
    ct0 = (((0,), (0,)), ((), ()))
    nt = (((1,), (1,)), ((), ()))
    mqk_ref[...] = lax.dot_general(wq_ref[...], wk_ref[...], ct0,
                                   preferred_element_type=jnp.float32,
                                   precision=_COR)
    wvd_ref[...] = jnp.dot(wd_ref[...], wv_ref[...],
                           preferred_element_type=jnp.float32, precision=_FAST)
    bias_ref[...] = lax.dot_general(bv_ref[...], wd_ref[...], nt,
                                    preferred_element_type=jnp.float32,
                                    precision=_FAST) + bd_ref[...]


def _prep(wq, wk, wv, wd, bv, bd):
    return pl.pallas_call(
        _prep_body,
        out_shape=[jax.ShapeDtypeStruct((DM, DM), jnp.float32),
                   jax.ShapeDtypeStruct((DM, DM), jnp.float32),
                   jax.ShapeDtypeStruct((1, DM), jnp.float32)],
        interpret=_INTERP,
    )(wq, wk, wv, wd, bv.reshape(1, DM), bd.reshape(1, DM))


def _a_body(trig_ref, x_ref, o_ref):
    o_ref[0, 0] = jnp.dot(trig_ref[0], x_ref[0],
                          preferred_element_type=jnp.float32, precision=_COR)


def _stage_a(x):
    return pl.pallas_call(
        _a_body,
        grid=(B, 2),
        in_specs=[
            pl.BlockSpec((1, FP, L), lambda b, s: (s, 0, 0)),
            pl.BlockSpec((1, L, DM), lambda b, s: (b, 0, 0)),
        ],
        out_specs=pl.BlockSpec((1, 1, FP, DM), lambda b, s: (b, s, 0, 0)),
        out_shape=jax.ShapeDtypeStruct((B, 2, FP, DM), jnp.float32),
        interpret=_INTERP,
    )(_TRIG, x)


def _b_body(xcs_ref, mqk_ref, wcs_ref, o_ref):
    xc = xcs_ref[0, 0]
    xs = xcs_ref[0, 1]
    t1 = jnp.dot(xc, mqk_ref[...],
                 preferred_element_type=jnp.float32, precision=_COR)
    t2 = jnp.dot(xs, mqk_ref[...],
                 preferred_element_type=jnp.float32, precision=_COR)
    sr = jnp.sum(t1 * xc + t2 * xs, axis=1)
    si = jnp.sum(t1 * xs - t2 * xc, axis=1)
    srsi = jnp.concatenate([sr, si])[None, :]
    o_ref[0] = jnp.dot(srsi, wcs_ref[...],
                       preferred_element_type=jnp.float32, precision=_COR)


def _stage_b(xcs, mqk):
    mv3 = pl.pallas_call(
        _b_body,
        grid=(B,),
        in_specs=[
            pl.BlockSpec((1, 2, FP, DM), lambda b: (b, 0, 0, 0)),
            pl.BlockSpec((DM, DM), lambda b: (0, 0)),
            pl.BlockSpec((2 * FP, L), lambda b: (0, 0)),
        ],
        out_specs=pl.BlockSpec((1, 1, L), lambda b: (b, 0, 0)),
        out_shape=jax.ShapeDtypeStruct((B, 1, L), jnp.float32),
        interpret=_INTERP,
    )(xcs, mqk, _WCS)
    return mv3.reshape(B, L)


def _sc_body(mv_hbm, idx_hbm, w_hbm, mv_v, m_v, red_f, red_i, idx_v, w_v):
    cid = lax.axis_index("c")
    sid = lax.axis_index("s")

    @pl.when(jnp.logical_and(cid == 0, sid == 0))
    def _():
        pltpu.sync_copy(mv_hbm, mv_v)
        lanes = lax.iota(jnp.int32, 16)
        one_lane = lanes == 0

        def mloop(j, carry):
            acc = (mv_v[0, pl.ds(j * 16, 16)] + mv_v[1, pl.ds(j * 16, 16)] +
                   mv_v[2, pl.ds(j * 16, 16)] + mv_v[3, pl.ds(j * 16, 16)])
            m_v[pl.ds(j * 16, 16)] = acc * 0.25
            return carry

        lax.fori_loop(0, L // 16, mloop, 0)

        def rot_argmax(val, idx):
            for s in (1, 2, 4, 8):
                red_f[pl.ds(0, 16)] = val
                red_f[pl.ds(16, 16)] = val
                red_i[pl.ds(0, 16)] = idx
                red_i[pl.ds(16, 16)] = idx
                rv = red_f[pl.ds(s, 16)]
                ri = red_i[pl.ds(s, 16)]
                cond = jnp.logical_or(rv > val,
                                      jnp.logical_and(rv == val, ri < idx))
                val = jnp.where(cond, rv, val)
                idx = jnp.where(cond, ri, idx)
            return val, idx

        def rot_sum(val):
            for s in (1, 2, 4, 8):
                red_f[pl.ds(0, 16)] = val
                red_f[pl.ds(16, 16)] = val
                val = val + red_f[pl.ds(s, 16)]
            return val

        idxv = jnp.zeros((16,), jnp.int32)
        taken = []
        wvecs = [jnp.zeros((16,), jnp.float32) for _ in range(B)]
        for r in range(TOPK):
            def scan(j, carry):
                mx, ix = carry
                vv = m_v[pl.ds(j * 16, 16)]
                iv = lanes + j * 16
                for c in taken:
                    vv = jnp.where(iv == c, -1e30, vv)
                sel = vv > mx
                return (jnp.where(sel, vv, mx), jnp.where(sel, iv, ix))

            mx, ix = lax.fori_loop(
                0, L // 16, scan,
                (jnp.full((16,), -1e30, jnp.float32), jnp.zeros((16,), jnp.int32)))
            _, chosen = rot_argmax(mx, ix)
            taken.append(chosen)
            idxv = jnp.where(lanes == r, chosen, idxv)

            def extract(j, carry):
                iv = lanes + j * 16
                hit = iv == chosen
                return tuple(
                    acc + jnp.where(hit, mv_v[b, pl.ds(j * 16, 16)], 0.0)
                    for b, acc in enumerate(carry))

            vals = lax.fori_loop(
                0, L // 16, extract,
                tuple(jnp.zeros((16,), jnp.float32) for _ in range(B)))
            for b in range(B):
                vb = rot_sum(vals[b])
                wvecs[b] = jnp.where(lanes == r, vb, wvecs[b])

        idx_v[...] = idxv
        valid = lanes < TOPK
        for b in range(B):
            g = wvecs[b]
            gmx, _ = rot_argmax(jnp.where(valid, g, -1e30), lanes)
            e = jnp.where(valid, jnp.exp(g - gmx), 0.0)
            s = rot_sum(e)
            w_v[b, :] = e / s
        pltpu.sync_copy(idx_v, idx_hbm)
        pltpu.sync_copy(w_v, w_hbm)


@functools.cache
def _sc_topk_fn():
    return functools.partial(
        pl.kernel,
        out_type=[jax.ShapeDtypeStruct((16,), jnp.int32),
                  jax.ShapeDtypeStruct((B, 16), jnp.float32)],
        mesh=plsc.VectorSubcoreMesh(core_axis_name="c", subcore_axis_name="s"),
        scratch_types=[pltpu.VMEM((B, L), jnp.float32),
                       pltpu.VMEM((L,), jnp.float32),
                       pltpu.VMEM((32,), jnp.float32),
                       pltpu.VMEM((32,), jnp.int32),
                       pltpu.VMEM((16,), jnp.int32),
                       pltpu.VMEM((B, 16), jnp.float32)],
    )(_sc_body)


def _sc_topk(mv):
    return _sc_topk_fn()(mv)


GRT = 512


def _e_body(x_ref, wvd_ref, bias_ref, g_ref, beta_ref, w_ref, idx_ref,
            o_ref):
    b = pl.program_id(0)
    nt = (((1,), (1,)), ((), ()))
    u = lax.dot_general(x_ref[0], wvd_ref[...], nt,
                        preferred_element_type=jnp.float32, precision=_FAST)
    for rt in range(L // GRT):
        row = lax.broadcasted_iota(jnp.int32, (GRT, L), 0) + rt * GRT
        col = lax.broadcasted_iota(jnp.int32, (GRT, L), 1)
        delta = (col - row) & (L - 1)
        gmat = jnp.zeros((GRT, L), jnp.float32)
        for i in range(TOPK):
            gmat = jnp.where(delta == idx_ref[i], w_ref[b, i], gmat)
        ctx = jnp.dot(gmat, u, preferred_element_type=jnp.float32,
                      precision=_FAST)
        rs = slice(rt * GRT, (rt + 1) * GRT)
        h = ctx + bias_ref[...] + x_ref[0, rs, :]
        mu = jnp.mean(h, axis=1, keepdims=True)
        d = h - mu
        var = jnp.mean(d * d, axis=1, keepdims=True)
        o_ref[0, rs, :] = (d * lax.rsqrt(var + EPS) * g_ref[...]
                           + beta_ref[...])


def _stage_e(x, wvd, bias, ln_g, ln_b, w44, idx16):
    return pl.pallas_call(
        _e_body,
        grid=(B,),
        in_specs=[
            pl.BlockSpec((1, L, DM), lambda b: (b, 0, 0)),
            pl.BlockSpec((DM, DM), lambda b: (0, 0)),
            pl.BlockSpec((1, DM), lambda b: (0, 0)),
            pl.BlockSpec((1, DM), lambda b: (0, 0)),
            pl.BlockSpec((1, DM), lambda b: (0, 0)),
            pl.BlockSpec(memory_space=pltpu.SMEM),
            pl.BlockSpec(memory_space=pltpu.SMEM),
        ],
        out_specs=pl.BlockSpec((1, L, DM), lambda b: (b, 0, 0)),
        out_shape=jax.ShapeDtypeStruct((B, L, DM), jnp.float32),
        interpret=_INTERP,
    )(x, wvd, bias, ln_g.reshape(1, DM), ln_b.reshape(1, DM), w44, idx16)


def kernel(input_tensor, attention_mask, Wq, bq, Wk, bk, Wv, bv, Wd, bd,
           ln_g, ln_b):
    del attention_mask, bq, bk
    x = input_tensor
    mqk, wvd, bias = _prep(Wq, Wk, Wv, Wd, bv, bd)
    xcs = _stage_a(x)
    mv = _stage_b(xcs, mqk)
    idx16, w44 = _sc_topk(mv)
    return _stage_e(x, wvd, bias, ln_g, ln_b, w44, idx16)


# speedup vs baseline: 1.0523x; 1.0523x over previous
import functools
import math

import jax
import jax.numpy as jnp
import numpy as np
from jax import lax
from jax.experimental import pallas as pl
from jax.experimental.pallas import tpu as pltpu
from jax.experimental.pallas import tpu_sc as plsc

B, L, DM, H = 4, 2048, 1024, 16
LEFT, RIGHT = 410, 1025
F = RIGHT - LEFT
FP = 640
TOPK = int(1 * math.log(L))
EPS = 1e-12
CBLK = 256
_INTERP = False

_HI = jax.lax.Precision.HIGHEST
_COR = jax.lax.Precision.HIGHEST
_FAST = jax.lax.Precision.DEFAULT


def _build_consts():
    t = np.arange(L, dtype=np.int64)
    f = np.arange(FP, dtype=np.int64) + LEFT
    ang = 2.0 * np.pi * ((f[:, None] * t[None, :]) % L) / L
    cst = np.cos(ang)
    snt = np.sin(ang)
    cst[F:, :] = 0.0
    snt[F:, :] = 0.0
    coef = np.zeros((FP, 1))
    coef[:F] = 2.0
    coef[F - 1] = 1.0
    wcm = coef * np.cos(ang) / (L * DM)
    wsm = -coef * np.sin(ang) / (L * DM)
    wsm[F - 1, :] = 0.0
    wcm[F:, :] = 0.0
    trig = np.stack([cst, snt]).astype(np.float32)
    wcs = np.concatenate([wcm, wsm], 0).astype(np.float32)
    return trig, wcs


_TRIG, _WCS = _build_consts()


def _prep1_body(wq_ref, wk_ref, mqk_ref):
    ct0 = (((0,), (0,)), ((), ()))
    mqk_ref[...] = lax.dot_general(wq_ref[...], wk_ref[...], ct0,
                                   preferred_element_type=jnp.float32,
                                   precision=_COR)


def _prep1(wq, wk):
    return pl.pallas_call(
        _prep1_body,
        out_shape=jax.ShapeDtypeStruct((DM, DM), jnp.float32),
        interpret=_INTERP,
    )(wq, wk)


def _prep2_body(wv_ref, wd_ref, bv_ref, bd_ref, wvd_ref, bias_ref):
    nt = (((1,), (1,)), ((), ()))
    wvd_ref[...] = jnp.dot(wd_ref[...], wv_ref[...],
                           preferred_element_type=jnp.float32, precision=_FAST)
    bias_ref[...] = lax.dot_general(bv_ref[...], wd_ref[...], nt,
                                    preferred_element_type=jnp.float32,
                                    precision=_FAST) + bd_ref[...]


def _prep2(wv, wd, bv, bd):
    return pl.pallas_call(
        _prep2_body,
        out_shape=[jax.ShapeDtypeStruct((DM, DM), jnp.float32),
                   jax.ShapeDtypeStruct((1, DM), jnp.float32)],
        interpret=_INTERP,
    )(wv, wd, bv.reshape(1, DM), bd.reshape(1, DM))


def _a_body(trig_ref, x_ref, o_ref):
    o_ref[0, 0] = jnp.dot(trig_ref[0], x_ref[0],
                          preferred_element_type=jnp.float32, precision=_COR)


def _stage_a(x):
    return pl.pallas_call(
        _a_body,
        grid=(B, 2),
        in_specs=[
            pl.BlockSpec((1, FP, L), lambda b, s: (s, 0, 0)),
            pl.BlockSpec((1, L, DM), lambda b, s: (b, 0, 0)),
        ],
        out_specs=pl.BlockSpec((1, 1, FP, DM), lambda b, s: (b, s, 0, 0)),
        out_shape=jax.ShapeDtypeStruct((B, 2, FP, DM), jnp.float32),
        interpret=_INTERP,
    )(_TRIG, x)


def _b_body(xcs_ref, mqk_ref, wcs_ref, o_ref):
    xc = xcs_ref[0, 0]
    xs = xcs_ref[0, 1]
    t1 = jnp.dot(xc, mqk_ref[...],
                 preferred_element_type=jnp.float32, precision=_COR)
    t2 = jnp.dot(xs, mqk_ref[...],
                 preferred_element_type=jnp.float32, precision=_COR)
    sr = jnp.sum(t1 * xc + t2 * xs, axis=1)
    si = jnp.sum(t1 * xs - t2 * xc, axis=1)
    srsi = jnp.concatenate([sr, si])[None, :]
    o_ref[0] = jnp.dot(srsi, wcs_ref[...],
                       preferred_element_type=jnp.float32, precision=_COR)


def _stage_b(xcs, mqk):
    mv3 = pl.pallas_call(
        _b_body,
        grid=(B,),
        in_specs=[
            pl.BlockSpec((1, 2, FP, DM), lambda b: (b, 0, 0, 0)),
            pl.BlockSpec((DM, DM), lambda b: (0, 0)),
            pl.BlockSpec((2 * FP, L), lambda b: (0, 0)),
        ],
        out_specs=pl.BlockSpec((1, 1, L), lambda b: (b, 0, 0)),
        out_shape=jax.ShapeDtypeStruct((B, 1, L), jnp.float32),
        interpret=_INTERP,
    )(xcs, mqk, _WCS)
    return mv3.reshape(B, L)


def _sc_body(mv_hbm, idx_hbm, w_hbm, mv_v, m_v, red_f, red_i, idx_v, w_v):
    cid = lax.axis_index("c")
    sid = lax.axis_index("s")

    @pl.when(jnp.logical_and(cid == 0, sid == 0))
    def _():
        pltpu.sync_copy(mv_hbm, mv_v)
        lanes = lax.iota(jnp.int32, 16)
        one_lane = lanes == 0

        def mloop(j, carry):
            acc = (mv_v[0, pl.ds(j * 16, 16)] + mv_v[1, pl.ds(j * 16, 16)] +
                   mv_v[2, pl.ds(j * 16, 16)] + mv_v[3, pl.ds(j * 16, 16)])
            m_v[pl.ds(j * 16, 16)] = acc * 0.25
            return carry

        lax.fori_loop(0, L // 16, mloop, 0)

        def rot_argmax(val, idx):
            for s in (1, 2, 4, 8):
                red_f[pl.ds(0, 16)] = val
                red_f[pl.ds(16, 16)] = val
                red_i[pl.ds(0, 16)] = idx
                red_i[pl.ds(16, 16)] = idx
                rv = red_f[pl.ds(s, 16)]
                ri = red_i[pl.ds(s, 16)]
                cond = jnp.logical_or(rv > val,
                                      jnp.logical_and(rv == val, ri < idx))
                val = jnp.where(cond, rv, val)
                idx = jnp.where(cond, ri, idx)
            return val, idx

        def rot_sum(val):
            for s in (1, 2, 4, 8):
                red_f[pl.ds(0, 16)] = val
                red_f[pl.ds(16, 16)] = val
                val = val + red_f[pl.ds(s, 16)]
            return val

        idxv = jnp.zeros((16,), jnp.int32)
        taken = []
        wvecs = [jnp.zeros((16,), jnp.float32) for _ in range(B)]
        for r in range(TOPK):
            def scan(j, carry):
                mx, ix = carry
                vv = m_v[pl.ds(j * 16, 16)]
                iv = lanes + j * 16
                for c in taken:
                    vv = jnp.where(iv == c, -1e30, vv)
                sel = vv > mx
                return (jnp.where(sel, vv, mx), jnp.where(sel, iv, ix))

            mx, ix = lax.fori_loop(
                0, L // 16, scan,
                (jnp.full((16,), -1e30, jnp.float32), jnp.zeros((16,), jnp.int32)),
                unroll=4)
            _, chosen = rot_argmax(mx, ix)
            taken.append(chosen)
            idxv = jnp.where(lanes == r, chosen, idxv)

            def extract(j, carry):
                iv = lanes + j * 16
                hit = iv == chosen
                return tuple(
                    acc + jnp.where(hit, mv_v[b, pl.ds(j * 16, 16)], 0.0)
                    for b, acc in enumerate(carry))

            vals = lax.fori_loop(
                0, L // 16, extract,
                tuple(jnp.zeros((16,), jnp.float32) for _ in range(B)),
                unroll=4)
            for b in range(B):
                vb = rot_sum(vals[b])
                wvecs[b] = jnp.where(lanes == r, vb, wvecs[b])

        idx_v[...] = idxv
        valid = lanes < TOPK
        for b in range(B):
            g = wvecs[b]
            gmx, _ = rot_argmax(jnp.where(valid, g, -1e30), lanes)
            e = jnp.where(valid, jnp.exp(g - gmx), 0.0)
            s = rot_sum(e)
            w_v[b, :] = e / s
        pltpu.sync_copy(idx_v, idx_hbm)
        pltpu.sync_copy(w_v, w_hbm)


@functools.cache
def _sc_topk_fn():
    return functools.partial(
        pl.kernel,
        out_type=[jax.ShapeDtypeStruct((16,), jnp.int32),
                  jax.ShapeDtypeStruct((B, 16), jnp.float32)],
        mesh=plsc.VectorSubcoreMesh(core_axis_name="c", subcore_axis_name="s"),
        scratch_types=[pltpu.VMEM((B, L), jnp.float32),
                       pltpu.VMEM((L,), jnp.float32),
                       pltpu.VMEM((32,), jnp.float32),
                       pltpu.VMEM((32,), jnp.int32),
                       pltpu.VMEM((16,), jnp.int32),
                       pltpu.VMEM((B, 16), jnp.float32)],
    )(_sc_body)


def _sc_topk(mv):
    return _sc_topk_fn()(mv)


GRT = 512


def _e_body(x_ref, wvd_ref, bias_ref, g_ref, beta_ref, w_ref, idx_ref,
            o_ref):
    b = pl.program_id(0)
    nt = (((1,), (1,)), ((), ()))
    u = lax.dot_general(x_ref[0], wvd_ref[...], nt,
                        preferred_element_type=jnp.float32, precision=_FAST)
    ub = u.astype(jnp.bfloat16)
    for rt in range(L // GRT):
        row = lax.broadcasted_iota(jnp.int32, (GRT, L), 0) + rt * GRT
        col = lax.broadcasted_iota(jnp.int32, (GRT, L), 1)
        delta = (col - row) & (L - 1)
        gmat = jnp.zeros((GRT, L), jnp.float32)
        for i in range(TOPK):
            gmat = jnp.where(delta == idx_ref[i], w_ref[b, i], gmat)
        ctx = jnp.dot(gmat.astype(jnp.bfloat16), ub,
                      preferred_element_type=jnp.float32, precision=_FAST)
        rs = slice(rt * GRT, (rt + 1) * GRT)
        h = ctx + bias_ref[...] + x_ref[0, rs, :]
        mu = jnp.mean(h, axis=1, keepdims=True)
        d = h - mu
        var = jnp.mean(d * d, axis=1, keepdims=True)
        o_ref[0, rs, :] = (d * lax.rsqrt(var + EPS) * g_ref[...]
                           + beta_ref[...])


def _stage_e(x, wvd, bias, ln_g, ln_b, w44, idx16):
    return pl.pallas_call(
        _e_body,
        grid=(B,),
        in_specs=[
            pl.BlockSpec((1, L, DM), lambda b: (b, 0, 0)),
            pl.BlockSpec((DM, DM), lambda b: (0, 0)),
            pl.BlockSpec((1, DM), lambda b: (0, 0)),
            pl.BlockSpec((1, DM), lambda b: (0, 0)),
            pl.BlockSpec((1, DM), lambda b: (0, 0)),
            pl.BlockSpec(memory_space=pltpu.SMEM),
            pl.BlockSpec(memory_space=pltpu.SMEM),
        ],
        out_specs=pl.BlockSpec((1, L, DM), lambda b: (b, 0, 0)),
        out_shape=jax.ShapeDtypeStruct((B, L, DM), jnp.float32),
        interpret=_INTERP,
    )(x, wvd, bias, ln_g.reshape(1, DM), ln_b.reshape(1, DM), w44, idx16)


def kernel(input_tensor, attention_mask, Wq, bq, Wk, bk, Wv, bv, Wd, bd,
           ln_g, ln_b):
    del attention_mask, bq, bk
    x = input_tensor
    mqk = _prep1(Wq, Wk)
    xcs = _stage_a(x)
    mv = _stage_b(xcs, mqk)
    idx16, w44 = _sc_topk(mv)
    wvd, bias = _prep2(Wv, Wd, bv, bd)
    return _stage_e(x, wvd, bias, ln_g, ln_b, w44, idx16)
